# baseline (speedup 1.0000x reference)
import jax
import jax.numpy as jnp
from jax.experimental import pallas as pl
from jax.experimental.pallas import tpu as pltpu

_TT = 512


def _moe_body(x_ref, wg_ref, bg_ref, we_ref, be_ref,
              out_ref, scores_ref, xs_ref, wef_ref, bep_ref, s2_ref):
    D = x_ref.shape[1]
    E = wg_ref.shape[1]
    TT = x_ref.shape[0]
    i = pl.program_id(0)

    @pl.when(i < E)
    def _cast_chunk():
        wef_ref[pl.ds(i * D, D), :] = we_ref[0].astype(jnp.bfloat16)

    @pl.when(i == 0)
    def _bias_pad():
        bep_ref[...] = jnp.concatenate(
            [be_ref[...].astype(jnp.bfloat16),
             jnp.zeros((128 - E, D), jnp.bfloat16)], axis=0)

    @pl.when(i >= E)
    def _tile():
        x = x_ref[...]
        logits = jnp.dot(x, wg_ref[...], preferred_element_type=jnp.float32)
        logits = logits + bg_ref[...]
        m = jnp.max(logits, axis=-1, keepdims=True)
        e = jnp.exp(logits - m)
        s = e / jnp.sum(e, axis=-1, keepdims=True)
        scores_ref[...] = s
        s2_ref[...] = jnp.concatenate([s] * (128 // E),
                                      axis=1).astype(jnp.bfloat16)
        H = TT // 4
        col = jax.lax.broadcasted_iota(jnp.int32, (H, E), 1)
        for h in range(4):
            r = pl.ds(h * H, H)
            sh = s[h * H:(h + 1) * H]
            xh = x[h * H:(h + 1) * H]
            for kk in range(E):
                s_kk = jnp.sum(jnp.where(col == kk, sh, 0.0), axis=1,
                               keepdims=True)
                xs_ref[r, kk * D:(kk + 1) * D] = (xh * s_kk).astype(
                    jnp.bfloat16)
            out_ref[r, :] = (
                jnp.dot(xs_ref[r, :], wef_ref[...],
                        preferred_element_type=jnp.float32)
                + jnp.dot(s2_ref[r, :], bep_ref[...],
                          preferred_element_type=jnp.float32)
            )


def kernel(x, Wg, bg, We, be):
    T, D = x.shape
    E = Wg.shape[1]
    n = T // _TT

    out, scores = pl.pallas_call(
        _moe_body,
        grid=(E + n,),
        in_specs=[
            pl.BlockSpec((_TT, D), lambda i: (jnp.maximum(i - E, 0), 0)),
            pl.BlockSpec((D, E), lambda i: (0, 0)),
            pl.BlockSpec((1, E), lambda i: (0, 0)),
            pl.BlockSpec((1, D, D),
                         lambda i: (jnp.minimum(i, E - 1), 0, 0)),
            pl.BlockSpec((E, D), lambda i: (0, 0)),
        ],
        out_specs=[
            pl.BlockSpec((_TT, D), lambda i: (jnp.maximum(i - E, 0), 0)),
            pl.BlockSpec((_TT, E), lambda i: (jnp.maximum(i - E, 0), 0)),
        ],
        out_shape=[
            jax.ShapeDtypeStruct((T, D), jnp.float32),
            jax.ShapeDtypeStruct((T, E), jnp.float32),
        ],
        scratch_shapes=[
            pltpu.VMEM((_TT, E * D), jnp.bfloat16),
            pltpu.VMEM((E * D, D), jnp.bfloat16),
            pltpu.VMEM((128, D), jnp.bfloat16),
            pltpu.VMEM((_TT, 128), jnp.bfloat16),
        ],
        compiler_params=pltpu.CompilerParams(
            dimension_semantics=("arbitrary",),
        ),
    )(x, Wg, bg.reshape(1, E), We, be)
    return out, scores
